# Initial kernel scaffold; baseline (speedup 1.0000x reference)
import jax
import jax.numpy as jnp
from jax.experimental import pallas as pl


def kernel(batch, neigh_samples, embedding, W_self_0, W_neigh_0, W_self_1, W_neigh_1, W_self_2, W_neigh_2):
    raise NotImplementedError("write your pallas kernel here")


# speedup vs baseline: 11.3656x; 11.3656x over previous
import functools

import jax
import jax.numpy as jnp
from jax import lax
from jax.experimental import pallas as pl
from jax.experimental.pallas import tpu as pltpu
from jax.experimental.pallas import tpu_sc as plsc

B = 16384
E = 128
NEIGH = 25
NL = 3
NW = 32
EPW = B // NW
CH = 4
ROWS = CH * NEIGH
NCH = EPW // CH
NVR = E // 16


def _sc_gather_mean(batch2d, neigh4d, embedding):
    mesh = plsc.VectorSubcoreMesh(core_axis_name="c", subcore_axis_name="s")
    n_self = EPW // 128

    @functools.partial(
        pl.kernel,
        out_type=(
            jax.ShapeDtypeStruct((B, E), jnp.float32),
            jax.ShapeDtypeStruct((NL, B, E), jnp.float32),
        ),
        mesh=mesh,
        scratch_types=[
            pltpu.VMEM((n_self, 128), jnp.int32),
            pltpu.VMEM((NCH, ROWS), jnp.int32),
            pltpu.VMEM((ROWS, E), jnp.float32),
            pltpu.VMEM((ROWS, E), jnp.float32),
            pltpu.VMEM((EPW, E), jnp.float32),
            pltpu.SemaphoreType.DMA,
            pltpu.SemaphoreType.DMA,
        ],
    )
    def k(batch_hbm, neigh_hbm, emb_hbm, out_self, out_sums,
          sidx_v, idx_v, rows0, rows1, out_v, sem0, sem1):
        wid = lax.axis_index("s") * 2 + lax.axis_index("c")
        base = wid * EPW
        bufs = (rows0, rows1)
        sems = (sem0, sem1)

        pltpu.sync_copy(batch_hbm.at[wid], sidx_v)
        for c in range(n_self):
            pltpu.async_copy(emb_hbm.at[sidx_v.at[c]],
                             out_v.at[pl.ds(c * 128, 128), :], sem0)
        for c in range(n_self):
            pltpu.make_async_copy(emb_hbm.at[sidx_v.at[c]],
                                  out_v.at[pl.ds(c * 128, 128), :], sem0).wait()
        pltpu.sync_copy(out_v, out_self.at[pl.ds(base, EPW), :])

        def accumulate(buf, c):
            for e in range(CH):
                r0 = e * NEIGH
                accs = tuple(buf[r0, pl.ds(r * 16, 16)] for r in range(NVR))

                def jbody(j, a):
                    return tuple(a[r] + buf[j, pl.ds(r * 16, 16)]
                                 for r in range(NVR))

                accs = lax.fori_loop(r0 + 1, r0 + NEIGH, jbody, accs)
                orow = c * CH + e
                for r in range(NVR):
                    out_v[orow, pl.ds(r * 16, 16)] = accs[r]

        for layer in range(NL):
            pltpu.sync_copy(neigh_hbm.at[layer, wid], idx_v)
            pltpu.async_copy(emb_hbm.at[idx_v.at[0]], rows0, sem0)
            pltpu.async_copy(emb_hbm.at[idx_v.at[1]], rows1, sem1)

            def pbody(p, _):
                for b in range(2):
                    c = 2 * p + b
                    pltpu.make_async_copy(emb_hbm.at[idx_v.at[c]],
                                          bufs[b], sems[b]).wait()
                    accumulate(bufs[b], c)

                    @pl.when(c + 2 < NCH)
                    def _():
                        pltpu.async_copy(emb_hbm.at[idx_v.at[c + 2]],
                                         bufs[b], sems[b])
                return 0

            lax.fori_loop(0, NCH // 2, pbody, 0)
            pltpu.sync_copy(out_v, out_sums.at[layer, pl.ds(base, EPW), :])

    return k(batch2d, neigh4d, embedding)


def _tc_mlp(self_vec, sums, ws0, wn0, ws1, wn1, ws2, wn2):
    TB = 2048
    grid = (B // TB,)

    def body(s_ref, m_ref, ws0r, wn0r, ws1r, wn1r, ws2r, wn2r, o_ref):
        h = jnp.maximum(
            jnp.dot(s_ref[0], ws0r[0], preferred_element_type=jnp.float32)
            + jnp.dot(m_ref[0, 0], wn0r[0], preferred_element_type=jnp.float32),
            0.0)
        h = jnp.maximum(
            jnp.dot(h, ws1r[0], preferred_element_type=jnp.float32)
            + jnp.dot(m_ref[0, 1], wn1r[0], preferred_element_type=jnp.float32),
            0.0)
        o_ref[0] = jnp.maximum(
            jnp.dot(h, ws2r[0], preferred_element_type=jnp.float32)
            + jnp.dot(m_ref[0, 2], wn2r[0], preferred_element_type=jnp.float32),
            0.0)

    def wspec(w):
        return pl.BlockSpec((1,) + w.shape, lambda i: (0, 0, 0))

    ws = [w[None] for w in (ws0, wn0, ws1, wn1, ws2, wn2)]
    return pl.pallas_call(
        body,
        grid=grid,
        in_specs=[
            pl.BlockSpec((1, TB, E), lambda i: (i, 0, 0)),
            pl.BlockSpec((1, NL, TB, E), lambda i: (i, 0, 0, 0)),
        ] + [wspec(w) for w in (ws0, wn0, ws1, wn1, ws2, wn2)],
        out_specs=pl.BlockSpec((1, TB, 512), lambda i: (i, 0, 0)),
        out_shape=jax.ShapeDtypeStruct((B // TB, TB, 512), jnp.float32),
    )(self_vec.reshape(B // TB, TB, E), sums, *ws).reshape(B, 512)


def kernel(batch, neigh_samples, embedding,
           W_self_0, W_neigh_0, W_self_1, W_neigh_1, W_self_2, W_neigh_2):
    batch2d = batch.reshape(NW, EPW // 128, 128)
    neigh4d = neigh_samples.reshape(NL, NW, NCH, ROWS)
    self_vec, sums = _sc_gather_mean(batch2d, neigh4d, embedding)
    TB = 2048
    sums_t = sums.reshape(NL, B // TB, TB, E).transpose(1, 0, 2, 3)
    inv = jnp.float32(1.0 / NEIGH)
    out = _tc_mlp(self_vec, sums_t,
                  W_self_0, W_neigh_0 * inv,
                  W_self_1, W_neigh_1 * inv,
                  W_self_2, W_neigh_2 * inv)
    return out
